# baseline (speedup 1.0000x reference)
import functools

import jax
import jax.numpy as jnp
from jax.experimental import pallas as pl
from jax.experimental.pallas import tpu as pltpu

B = 32
P = 26
D = 64
N = 1_000_000
K = 100

TN = 16384
S = 32
S2 = 1024
GRID = (N + TN - 1) // TN
NPAD = GRID * TN
C = NPAD // S
C2 = NPAD // S2

_I32MAX = jnp.iinfo(jnp.int32).max


def _score_kernel(query_ref, preds_ref, item_ref, pw_ref, scores_ref, cmax_ref):
    t = pl.program_id(0)
    dense = jax.lax.dot_general(
        query_ref[...], item_ref[...],
        dimension_numbers=(((1,), (1,)), ((), ())),
        preferred_element_type=jnp.float32,
    )
    pred = jax.lax.dot_general(
        preds_ref[...], pw_ref[...],
        dimension_numbers=(((1,), (0,)), ((), ())),
        preferred_element_type=jnp.float32,
    )
    scores = dense + pred
    limit = N - t * TN
    lane = jax.lax.broadcasted_iota(jnp.int32, (B, TN), 1)
    scores = jnp.where(lane < limit, scores, -jnp.inf)
    scores_ref[...] = scores
    cmax_ref[...] = jnp.max(scores.reshape(B, TN // S, S), axis=2)


def _topk_kernel(vals_ref, idx_ref, out_v_ref, out_i_ref, x_ref):
    x_ref[...] = vals_ref[...]
    gidx = idx_ref[...]
    lane = jax.lax.broadcasted_iota(jnp.int32, (B, 128), 1)

    def body(k, carry):
        acc_v, acc_i = carry
        x = x_ref[...]
        m = jnp.max(x, axis=1, keepdims=True)
        eq = x == m
        sel = jnp.min(jnp.where(eq, gidx, _I32MAX), axis=1, keepdims=True)
        x_ref[...] = jnp.where(gidx == sel, -jnp.inf, x)
        acc_v = jnp.where(lane == k, m, acc_v)
        acc_i = jnp.where(lane == k, sel, acc_i)
        return (acc_v, acc_i)

    out_v, out_i = jax.lax.fori_loop(
        0, K, body,
        (jnp.zeros((B, 128), jnp.float32), jnp.zeros((B, 128), jnp.int32)))
    out_v_ref[...] = out_v
    out_i_ref[...] = out_i


def _topk(vals, idx):
    n = vals.shape[1]
    out_v, out_i = pl.pallas_call(
        _topk_kernel,
        out_shape=[
            jax.ShapeDtypeStruct((B, 128), jnp.float32),
            jax.ShapeDtypeStruct((B, 128), jnp.int32),
        ],
        scratch_shapes=[pltpu.VMEM((B, n), jnp.float32)],
    )(vals, idx)
    return out_v[:, :K], out_i[:, :K]


@functools.partial(jax.jit, static_argnames=())
def kernel(pred_satisfied, query, item_embed, pred_weight):
    preds_f32 = pred_satisfied.astype(jnp.float32)
    scores, cmax = pl.pallas_call(
        _score_kernel,
        grid=(GRID,),
        in_specs=[
            pl.BlockSpec((B, D), lambda t: (0, 0)),
            pl.BlockSpec((B, P), lambda t: (0, 0)),
            pl.BlockSpec((TN, D), lambda t: (t, 0)),
            pl.BlockSpec((P, TN), lambda t: (0, t)),
        ],
        out_specs=[
            pl.BlockSpec((B, TN), lambda t: (0, t)),
            pl.BlockSpec((B, TN // S), lambda t: (0, t)),
        ],
        out_shape=[
            jax.ShapeDtypeStruct((B, NPAD), jnp.float32),
            jax.ShapeDtypeStruct((B, C), jnp.float32),
        ],
    )(query, preds_f32, item_embed, pred_weight)

    cmax2 = jnp.max(cmax.reshape(B, C2, S2 // S), axis=2)
    cmax2 = jnp.concatenate(
        [cmax2, jnp.full((B, 1024 - C2), -jnp.inf, jnp.float32)], axis=1)
    c2_iota = jnp.broadcast_to(jnp.arange(1024, dtype=jnp.int32), (B, 1024))
    _, coarse_ids = _topk(cmax2, c2_iota)

    fine_idx = (coarse_ids[:, :, None] * (S2 // S)
                + jnp.arange(S2 // S, dtype=jnp.int32)).reshape(B, K * S2 // S)
    fine_maxima = jnp.take_along_axis(cmax, fine_idx, axis=1)
    _, fine_ids = _topk(fine_maxima, fine_idx)

    cand_idx = (fine_ids[:, :, None] * S
                + jnp.arange(S, dtype=jnp.int32)).reshape(B, K * S)
    cand_vals = jnp.take_along_axis(scores, cand_idx, axis=1)
    top_vals, top_idx = _topk(cand_vals, cand_idx)
    return top_vals, top_idx


# speedup vs baseline: 1.0123x; 1.0123x over previous
import functools

import jax
import jax.numpy as jnp
from jax.experimental import pallas as pl
from jax.experimental.pallas import tpu as pltpu

B = 32
P = 26
D = 64
N = 1_000_000
K = 100

TN = 32768
S = 128
GRID = (N + TN - 1) // TN
NPAD = GRID * TN
C = NPAD // S

_I32MAX = jnp.iinfo(jnp.int32).max


def _score_kernel(query_ref, preds_ref, item_ref, pw_ref, scores_ref, cmax_ref):
    t = pl.program_id(0)
    dense = jax.lax.dot_general(
        query_ref[...], item_ref[...],
        dimension_numbers=(((1,), (1,)), ((), ())),
        preferred_element_type=jnp.float32,
    )
    pred = jax.lax.dot_general(
        preds_ref[...], pw_ref[...],
        dimension_numbers=(((1,), (0,)), ((), ())),
        preferred_element_type=jnp.float32,
    )
    scores = dense + pred
    limit = N - t * TN
    lane = jax.lax.broadcasted_iota(jnp.int32, (B, TN), 1)
    scores = jnp.where(lane < limit, scores, -jnp.inf)
    scores_ref[...] = scores
    cmax_ref[...] = jnp.max(scores.reshape(B, TN // S, S), axis=2)


def _topk_kernel(vals_ref, idx_ref, out_v_ref, out_i_ref, x_ref):
    x_ref[...] = vals_ref[...]
    gidx = idx_ref[...]
    lane = jax.lax.broadcasted_iota(jnp.int32, (B, 128), 1)

    def body(k, carry):
        acc_v, acc_i = carry
        x = x_ref[...]
        m = jnp.max(x, axis=1, keepdims=True)
        eq = x == m
        sel = jnp.min(jnp.where(eq, gidx, _I32MAX), axis=1, keepdims=True)
        x_ref[...] = jnp.where(gidx == sel, -jnp.inf, x)
        acc_v = jnp.where(lane == k, m, acc_v)
        acc_i = jnp.where(lane == k, sel, acc_i)
        return (acc_v, acc_i)

    out_v, out_i = jax.lax.fori_loop(
        0, K, body,
        (jnp.zeros((B, 128), jnp.float32), jnp.zeros((B, 128), jnp.int32)))
    out_v_ref[...] = out_v
    out_i_ref[...] = out_i


def _topk(vals, idx):
    n = vals.shape[1]
    if n % 128:
        pad = 128 - n % 128
        vals = jnp.concatenate(
            [vals, jnp.full((B, pad), -jnp.inf, vals.dtype)], axis=1)
        idx = jnp.concatenate(
            [idx, jnp.full((B, pad), _I32MAX, jnp.int32)], axis=1)
        n += pad
    out_v, out_i = pl.pallas_call(
        _topk_kernel,
        out_shape=[
            jax.ShapeDtypeStruct((B, 128), jnp.float32),
            jax.ShapeDtypeStruct((B, 128), jnp.int32),
        ],
        scratch_shapes=[pltpu.VMEM((B, n), jnp.float32)],
    )(vals, idx)
    return out_v[:, :K], out_i[:, :K]


@functools.partial(jax.jit, static_argnames=())
def kernel(pred_satisfied, query, item_embed, pred_weight):
    preds_f32 = pred_satisfied.astype(jnp.float32)
    scores, cmax = pl.pallas_call(
        _score_kernel,
        grid=(GRID,),
        in_specs=[
            pl.BlockSpec((B, D), lambda t: (0, 0)),
            pl.BlockSpec((B, P), lambda t: (0, 0)),
            pl.BlockSpec((TN, D), lambda t: (t, 0)),
            pl.BlockSpec((P, TN), lambda t: (0, t)),
        ],
        out_specs=[
            pl.BlockSpec((B, TN), lambda t: (0, t)),
            pl.BlockSpec((B, TN // S), lambda t: (0, t)),
        ],
        out_shape=[
            jax.ShapeDtypeStruct((B, NPAD), jnp.float32),
            jax.ShapeDtypeStruct((B, C), jnp.float32),
        ],
    )(query, preds_f32, item_embed, pred_weight)

    cm8 = jnp.max(cmax.reshape(B, C // 8, 8), axis=2)
    g_iota = jnp.broadcast_to(jnp.arange(C // 8, dtype=jnp.int32), (B, C // 8))
    _, g_ids = _topk(cm8, g_iota)

    chunk_cand = (g_ids[:, :, None] * 8
                  + jnp.arange(8, dtype=jnp.int32)).reshape(B, K * 8)
    chunk_maxima = jnp.take_along_axis(cmax, chunk_cand, axis=1)
    _, chunk_ids = _topk(chunk_maxima, chunk_cand)

    cand_idx = (chunk_ids[:, :, None] * S
                + jnp.arange(S, dtype=jnp.int32)).reshape(B, K * S)
    cand_vals = jnp.take_along_axis(scores, cand_idx, axis=1)

    fm = jnp.max(cand_vals.reshape(B, K * 4, 32), axis=2)
    fbase = (chunk_ids[:, :, None] * S
             + jnp.arange(0, S, 32, dtype=jnp.int32)).reshape(B, K * 4)
    _, sel_base = _topk(fm, fbase)

    final_idx = (sel_base[:, :, None]
                 + jnp.arange(32, dtype=jnp.int32)).reshape(B, K * 32)
    final_vals = jnp.take_along_axis(scores, final_idx, axis=1)
    top_vals, top_idx = _topk(final_vals, final_idx)
    return top_vals, top_idx


# speedup vs baseline: 1.0630x; 1.0501x over previous
import functools

import jax
import jax.numpy as jnp
from jax.experimental import pallas as pl
from jax.experimental.pallas import tpu as pltpu

B = 32
P = 26
D = 64
N = 1_000_000
K = 100

TN = 32768
S = 128
GRID = (N + TN - 1) // TN
NPAD = GRID * TN
C = NPAD // S
CT = TN // S

_I32MAX = jnp.iinfo(jnp.int32).max


def _extract_topk(x_ref, gidx, out_i_ref):
    lane = jax.lax.broadcasted_iota(jnp.int32, (B, 128), 1)

    def body(k, acc_i):
        x = x_ref[...]
        m = jnp.max(x, axis=1, keepdims=True)
        sel = jnp.min(jnp.where(x == m, gidx, _I32MAX), axis=1, keepdims=True)
        x_ref[...] = jnp.where(gidx == sel, -jnp.inf, x)
        return jnp.where(lane == k, sel, acc_i)

    out_i_ref[...] = jax.lax.fori_loop(
        0, K, body, jnp.zeros((B, 128), jnp.int32))


def _score_kernel(query_ref, preds_ref, item_ref, pw_ref,
                  scores_ref, cids_ref, cm_acc_ref):
    t = pl.program_id(0)

    @pl.when(t < GRID)
    def _():
        dense = jax.lax.dot_general(
            query_ref[...], item_ref[...],
            dimension_numbers=(((1,), (1,)), ((), ())),
            preferred_element_type=jnp.float32,
        )
        pred = jax.lax.dot_general(
            preds_ref[...], pw_ref[...],
            dimension_numbers=(((1,), (0,)), ((), ())),
            preferred_element_type=jnp.float32,
        )
        scores = dense + pred
        limit = N - t * TN
        lane = jax.lax.broadcasted_iota(jnp.int32, (B, TN), 1)
        scores = jnp.where(lane < limit, scores, -jnp.inf)
        scores_ref[...] = scores
        cm_acc_ref[:, pl.ds(t * CT, CT)] = jnp.max(
            scores.reshape(B, CT, S), axis=2)

    @pl.when(t == GRID)
    def _():
        cidx = jax.lax.broadcasted_iota(jnp.int32, (B, C), 1)
        _extract_topk(cm_acc_ref, cidx, cids_ref)


def _topk_kernel(vals_ref, idx_ref, out_v_ref, out_i_ref, x_ref):
    x_ref[...] = vals_ref[...]
    gidx = idx_ref[...]
    lane = jax.lax.broadcasted_iota(jnp.int32, (B, 128), 1)

    def body(k, carry):
        acc_v, acc_i = carry
        x = x_ref[...]
        m = jnp.max(x, axis=1, keepdims=True)
        sel = jnp.min(jnp.where(x == m, gidx, _I32MAX), axis=1, keepdims=True)
        x_ref[...] = jnp.where(gidx == sel, -jnp.inf, x)
        return (jnp.where(lane == k, m, acc_v), jnp.where(lane == k, sel, acc_i))

    out_v, out_i = jax.lax.fori_loop(
        0, K, body,
        (jnp.zeros((B, 128), jnp.float32), jnp.zeros((B, 128), jnp.int32)))
    out_v_ref[...] = out_v
    out_i_ref[...] = out_i


@functools.partial(jax.jit, static_argnames=())
def kernel(pred_satisfied, query, item_embed, pred_weight):
    preds_f32 = pred_satisfied.astype(jnp.float32)
    clamp = lambda t: jnp.minimum(t, GRID - 1)
    scores, cids = pl.pallas_call(
        _score_kernel,
        grid=(GRID + 1,),
        in_specs=[
            pl.BlockSpec((B, D), lambda t: (0, 0)),
            pl.BlockSpec((B, P), lambda t: (0, 0)),
            pl.BlockSpec((TN, D), lambda t: (clamp(t), 0)),
            pl.BlockSpec((P, TN), lambda t: (0, clamp(t))),
        ],
        out_specs=[
            pl.BlockSpec((B, TN), lambda t: (0, clamp(t))),
            pl.BlockSpec((B, 128), lambda t: (0, 0)),
        ],
        out_shape=[
            jax.ShapeDtypeStruct((B, NPAD), jnp.float32),
            jax.ShapeDtypeStruct((B, 128), jnp.int32),
        ],
        scratch_shapes=[pltpu.VMEM((B, C), jnp.float32)],
    )(query, preds_f32, item_embed, pred_weight)

    chunk_ids = cids[:, :K]
    cand_idx = (chunk_ids[:, :, None] * S
                + jnp.arange(S, dtype=jnp.int32)[None, None, :]).reshape(B, K * S)
    cand_vals = jnp.take_along_axis(scores, cand_idx, axis=1)
    out_v, out_i = pl.pallas_call(
        _topk_kernel,
        out_shape=[
            jax.ShapeDtypeStruct((B, 128), jnp.float32),
            jax.ShapeDtypeStruct((B, 128), jnp.int32),
        ],
        scratch_shapes=[pltpu.VMEM((B, K * S), jnp.float32)],
    )(cand_vals, cand_idx)
    return out_v[:, :K], out_i[:, :K]
